# baseline (speedup 1.0000x reference)
import jax
import jax.numpy as jnp
from jax.experimental import pallas as pl
from jax.experimental.pallas import tpu as pltpu

BATCH_TILE = 512


def _fused_mlp_kernel(x_ref, w0_ref, m0_ref, b0_ref, w1_ref, m1_ref, b1_ref,
                      o_ref, wm0_ref, wm1_ref):
    @pl.when(pl.program_id(0) == 0)
    def _prep_weights():
        wm0_ref[:] = (w0_ref[:] * m0_ref[:].astype(jnp.float32)).astype(
            jnp.bfloat16)
        wm1_ref[:] = (w1_ref[:] * m1_ref[:].astype(jnp.float32)).astype(
            jnp.bfloat16)

    xb = x_ref[:].astype(jnp.bfloat16)
    h = jax.lax.dot_general(
        xb, wm0_ref[:], (((1,), (1,)), ((), ())),
        preferred_element_type=jnp.float32)
    h = jnp.maximum(h + b0_ref[:], 0.0).astype(jnp.bfloat16)
    o_ref[:] = jax.lax.dot_general(
        h, wm1_ref[:], (((1,), (1,)), ((), ())),
        preferred_element_type=jnp.float32) + b1_ref[:]


def kernel(x, W0, b0, W1, b1, mask0, mask1):
    B, D0 = x.shape
    D1 = W0.shape[0]
    D2 = W1.shape[0]
    m0 = mask0.astype(jnp.int8)
    m1 = mask1.astype(jnp.int8)
    b0r = b0.reshape(1, D1)
    b1r = b1.reshape(1, D2)
    grid = (B // BATCH_TILE,)
    return pl.pallas_call(
        _fused_mlp_kernel,
        grid=grid,
        in_specs=[
            pl.BlockSpec((BATCH_TILE, D0), lambda i: (i, 0)),
            pl.BlockSpec((D1, D0), lambda i: (0, 0)),
            pl.BlockSpec((D1, D0), lambda i: (0, 0)),
            pl.BlockSpec((1, D1), lambda i: (0, 0)),
            pl.BlockSpec((D2, D1), lambda i: (0, 0)),
            pl.BlockSpec((D2, D1), lambda i: (0, 0)),
            pl.BlockSpec((1, D2), lambda i: (0, 0)),
        ],
        out_specs=pl.BlockSpec((BATCH_TILE, D2), lambda i: (i, 0)),
        out_shape=jax.ShapeDtypeStruct((B, D2), jnp.float32),
        scratch_shapes=[
            pltpu.VMEM((D1, D0), jnp.bfloat16),
            pltpu.VMEM((D2, D1), jnp.bfloat16),
        ],
    )(x, W0, m0, b0r, W1, m1, b1r)
